# baseline (speedup 1.0000x reference)
import functools

import jax
import jax.numpy as jnp
from jax import lax
from jax.experimental import pallas as pl
from jax.experimental.pallas import tpu as pltpu
from jax.experimental.pallas import tpu_sc as plsc

N = 10000
E = 320000
D_IN = 9
H = 128
G = 8

NC = 2
NS = 16
NW = NC * NS

NPAD = 10240
RPT = NPAD // NS
CH = 128
NCHUNK = 80
EPW = NCHUNK * CH
EPAD = NW * EPW

BLK = 2048
NBLK = NPAD // BLK

_f32 = jnp.float32


HRNG = NPAD // 2


def _make_sc_agg_body(w_cols, fill_ones):
    def body(h_hbm, src_hbm, dst_hbm, dep_hbm, out_hbm,
             src_v, dst_v, rows_v, acc_sh, sem0, sem1):
        del dep_hbm
        c = lax.axis_index("c")
        s = lax.axis_index("s")
        w = c * NS + s

        def zrow(j, _):
            for k in range(w_cols // 16):
                rows_v[0, j, pl.ds(k * 16, 16)] = jnp.zeros((16,), _f32)
            return 0
        lax.fori_loop(0, CH, zrow, 0)
        for k in range(RPT // CH):
            pltpu.sync_copy(rows_v.at[0],
                            acc_sh.at[pl.ds(s * RPT + k * CH, CH)])

        plsc.subcore_barrier()

        HALF = NCHUNK // 2
        if fill_ones:
            def orow(j, _):
                for k in range(w_cols // 16):
                    rows_v[0, j, pl.ds(k * 16, 16)] = jnp.ones((16,), _f32)
                return 0
            lax.fori_loop(0, CH, orow, 0)

        for half in range(2):
            pltpu.sync_copy(src_hbm.at[w, pl.ds(half * HALF, HALF)], src_v)
            pltpu.sync_copy(dst_hbm.at[w, pl.ds(half * HALF, HALF)], dst_v)

            if fill_ones:
                def step1(j, _):
                    pltpu.sync_copy(rows_v.at[0], acc_sh.at[dst_v.at[j]],
                                    add=True)
                    return 0
                lax.fori_loop(0, HALF, step1, 0)
                continue

            pltpu.async_copy(h_hbm.at[src_v.at[0]], rows_v.at[0], sem0)

            def step(jj, _):
                j0 = 2 * jj
                j1 = j0 + 1
                pltpu.async_copy(h_hbm.at[src_v.at[j1]], rows_v.at[1], sem1)
                pltpu.make_async_copy(
                    h_hbm.at[src_v.at[j0]], rows_v.at[0], sem0).wait()
                pltpu.sync_copy(rows_v.at[0], acc_sh.at[dst_v.at[j0]],
                                add=True)

                @pl.when(jj < HALF // 2 - 1)
                def _():
                    pltpu.async_copy(h_hbm.at[src_v.at[j0 + 2]],
                                     rows_v.at[0], sem0)

                pltpu.make_async_copy(
                    h_hbm.at[src_v.at[j1]], rows_v.at[1], sem1).wait()
                pltpu.sync_copy(rows_v.at[1], acc_sh.at[dst_v.at[j1]],
                                add=True)
                return 0

            lax.fori_loop(0, HALF // 2, step, 0)

        plsc.subcore_barrier()

        for k in range(RPT // CH):
            rows = pl.ds(s * RPT + k * CH, CH)
            pltpu.sync_copy(acc_sh.at[rows], out_hbm.at[c, rows])

    return body


def _sc_agg(table, src_slabs, dst_slabs, dep, w_cols, fill_ones=False):
    mesh = plsc.VectorSubcoreMesh(
        core_axis_name="c", subcore_axis_name="s", num_cores=NC,
        num_subcores=NS)
    kern = pl.kernel(
        _make_sc_agg_body(w_cols, fill_ones),
        out_type=jax.ShapeDtypeStruct((NC, NPAD, w_cols), _f32),
        mesh=mesh,
        scratch_types=[
            pltpu.VMEM((NCHUNK // 2, CH), jnp.int32),
            pltpu.VMEM((NCHUNK // 2, CH), jnp.int32),
            pltpu.VMEM((2, CH, w_cols), _f32),
            pltpu.VMEM_SHARED((NPAD, w_cols), _f32),
            pltpu.SemaphoreType.DMA,
            pltpu.SemaphoreType.DMA,
        ],
    )
    return kern(table, src_slabs, dst_slabs, dep)


def _ln_relu(y, g, be):
    mu = jnp.mean(y, axis=1, keepdims=True)
    var = jnp.mean((y - mu) ** 2, axis=1, keepdims=True)
    yn = (y - mu) * lax.rsqrt(var + 1e-5) * g + be
    return jnp.maximum(yn, 0.0)


def _tc_in_body(x_ref, w_ref, b_ref, g_ref, be_ref, o_ref):
    y = jnp.dot(x_ref[...], w_ref[...], preferred_element_type=_f32)
    o_ref[...] = _ln_relu(y + b_ref[...], g_ref[...], be_ref[...])


def _tc_input(x_pad, w0t, b0, g0, be0):
    row = lambda i: (i, 0)
    fixed = lambda i: (0, 0)
    return pl.pallas_call(
        _tc_in_body,
        grid=(NBLK,),
        in_specs=[
            pl.BlockSpec((BLK, H), row),
            pl.BlockSpec((H, H), fixed),
            pl.BlockSpec((1, H), fixed),
            pl.BlockSpec((1, H), fixed),
            pl.BlockSpec((1, H), fixed),
        ],
        out_specs=pl.BlockSpec((BLK, H), row),
        out_shape=jax.ShapeDtypeStruct((NPAD, H), _f32),
    )(x_pad, w0t, b0, g0, be0)


def _tc_rdeg_body(deg_ref, o_ref):
    d = deg_ref[0, :, :1] + deg_ref[1, :, :1]
    o_ref[...] = 1.0 / jnp.maximum(d, 1.0)


def _tc_rdeg(deg16):
    return pl.pallas_call(
        _tc_rdeg_body,
        grid=(NBLK,),
        in_specs=[pl.BlockSpec((NC, BLK, 16), lambda i: (0, i, 0))],
        out_specs=pl.BlockSpec((BLK, 1), lambda i: (i, 0)),
        out_shape=jax.ShapeDtypeStruct((NPAD, 1), _f32),
    )(deg16)


def _tc_layer_body(p_ref, rdeg_ref, h_ref, wl_ref, wr_ref, b_ref, g_ref,
                   be_ref, o_ref):
    agg = (p_ref[0] + p_ref[1]) * rdeg_ref[...]
    y = (jnp.dot(agg, wl_ref[...], preferred_element_type=_f32)
         + jnp.dot(h_ref[...].astype(_f32), wr_ref[...],
                   preferred_element_type=_f32)
         + b_ref[...])
    o_ref[...] = _ln_relu(y, g_ref[...], be_ref[...])


def _tc_layer(p, deg16, h, wlt, wrt, b, g, be):
    row = lambda i: (i, 0)
    fixed = lambda i: (0, 0)
    return pl.pallas_call(
        _tc_layer_body,
        grid=(NBLK,),
        in_specs=[
            pl.BlockSpec((NC, BLK, H), lambda i: (0, i, 0)),
            pl.BlockSpec((BLK, 1), lambda i: (i, 0)),
            pl.BlockSpec((BLK, H), row),
            pl.BlockSpec((H, H), fixed),
            pl.BlockSpec((H, H), fixed),
            pl.BlockSpec((1, H), fixed),
            pl.BlockSpec((1, H), fixed),
            pl.BlockSpec((1, H), fixed),
        ],
        out_specs=pl.BlockSpec((BLK, H), row),
        out_shape=jax.ShapeDtypeStruct((NPAD, H), _f32),
    )(p, deg16, h, wlt, wrt, b, g, be)


def _tc_final_body(p_ref, rdeg_ref, h_ref, wl_ref, wr_ref, b_ref, g_ref,
                   be_ref, batch_ref, o_ref, mean_ref, max_ref, cnt_ref):
    agg = (p_ref[0] + p_ref[1]) * rdeg_ref[...]
    y = (jnp.dot(agg, wl_ref[...], preferred_element_type=_f32)
         + jnp.dot(h_ref[...].astype(_f32), wr_ref[...],
                   preferred_element_type=_f32)
         + b_ref[...])
    ne = _ln_relu(y, g_ref[...], be_ref[...])
    o_ref[...] = ne

    i = pl.program_id(0)

    @pl.when(i == 0)
    def _():
        mean_ref[...] = jnp.zeros((G, H), _f32)
        cnt_ref[...] = jnp.zeros((G, H), _f32)
        max_ref[...] = jnp.full((G, H), -jnp.inf, _f32)

    b_ids = batch_ref[...]
    oh = (lax.broadcasted_iota(jnp.int32, (BLK, G), 1) == b_ids).astype(_f32)
    dn = (((0,), (0,)), ((), ()))
    mean_ref[...] += lax.dot_general(oh, ne, dn, preferred_element_type=_f32)
    cnt_ref[...] += lax.dot_general(oh, jnp.ones((BLK, H), _f32), dn,
                                    preferred_element_type=_f32)
    for gi in range(G):
        vg = jnp.where(b_ids == gi, ne, -jnp.inf)
        max_ref[gi:gi + 1, :] = jnp.maximum(
            max_ref[gi:gi + 1, :], jnp.max(vg, axis=0, keepdims=True))

    @pl.when(i == NBLK - 1)
    def _():
        mean_ref[...] = mean_ref[...] / jnp.maximum(cnt_ref[...], 1.0)


def _tc_final(p, deg16, h, wlt, wrt, b, g, be, batch_col):
    row = lambda i: (i, 0)
    fixed = lambda i: (0, 0)
    return pl.pallas_call(
        _tc_final_body,
        grid=(NBLK,),
        in_specs=[
            pl.BlockSpec((NC, BLK, H), lambda i: (0, i, 0)),
            pl.BlockSpec((BLK, 1), lambda i: (i, 0)),
            pl.BlockSpec((BLK, H), row),
            pl.BlockSpec((H, H), fixed),
            pl.BlockSpec((H, H), fixed),
            pl.BlockSpec((1, H), fixed),
            pl.BlockSpec((1, H), fixed),
            pl.BlockSpec((1, H), fixed),
            pl.BlockSpec((BLK, 1), row),
        ],
        out_specs=[
            pl.BlockSpec((BLK, H), row),
            pl.BlockSpec((G, H), fixed),
            pl.BlockSpec((G, H), fixed),
        ],
        out_shape=[
            jax.ShapeDtypeStruct((NPAD, H), _f32),
            jax.ShapeDtypeStruct((G, H), _f32),
            jax.ShapeDtypeStruct((G, H), _f32),
        ],
        scratch_shapes=[pltpu.VMEM((G, H), _f32)],
    )(p, deg16, h, wlt, wrt, b, g, be, batch_col)


def kernel(x, edge_index, batch, params):
    src = edge_index[0]
    dst = edge_index[1]

    per_w = E // NW
    pad_per_w = EPW - per_w
    pad_idx = jnp.broadcast_to(
        N + jnp.arange(pad_per_w, dtype=jnp.int32)[None, :], (NW, pad_per_w))
    src_slabs = jnp.concatenate(
        [src.reshape(NW, per_w), pad_idx], axis=1).reshape(NW, NCHUNK, CH)
    dst_slabs = jnp.concatenate(
        [dst.reshape(NW, per_w), pad_idx], axis=1).reshape(NW, NCHUNK, CH)

    x_pad = jnp.zeros((NPAD, H), _f32).at[:N, :D_IN].set(x)
    batch_col = jnp.full((NPAD, 1), G, jnp.int32).at[:N, 0].set(batch)

    p = params
    w0t = jnp.zeros((H, H), _f32).at[:D_IN, :].set(p['W0'].T)
    r2 = lambda v: v.reshape(1, H)

    ones_tab = jnp.ones((NPAD, H), _f32)
    deg16 = _sc_agg(ones_tab, src_slabs, dst_slabs, dst_slabs, H,
                    fill_ones=True)[:, :, :16]
    rdeg_col = _tc_rdeg(deg16)

    h = _tc_input(x_pad, w0t, r2(p['b0']), r2(p['g0']), r2(p['be0']))

    for li in ('1', '2'):
        agg = _sc_agg(h, src_slabs, dst_slabs, rdeg_col, H)
        h = _tc_layer(agg, rdeg_col, h, p['Wl' + li].T, p['Wr' + li].T,
                      r2(p['bl' + li]), r2(p['g' + li]), r2(p['be' + li]))

    agg = _sc_agg(h, src_slabs, dst_slabs, rdeg_col, H)
    node_pad, h_mean, h_max = _tc_final(
        agg, rdeg_col, h, p['Wl3'].T, p['Wr3'].T,
        r2(p['bl3']), r2(p['g3']), r2(p['be3']), batch_col)

    node_embed = node_pad[:N]
    graph_embed = jnp.concatenate([h_mean, h_max], axis=-1)
    return node_embed, graph_embed


# speedup vs baseline: 1.0071x; 1.0071x over previous
import jax
import jax.numpy as jnp
from jax import lax
from jax.experimental import pallas as pl
from jax.experimental.pallas import tpu as pltpu
from jax.experimental.pallas import tpu_sc as plsc

N = 10000
E = 320000
D_IN = 9
H = 128
G = 8

NC = 2
NS = 16
NW = NC * NS

NPAD = 10240
RPT = NPAD // NS
CH = 128
NCHUNK = 80
EPW = NCHUNK * CH
EPAD = NW * EPW

BLK = 2048
NBLK = NPAD // BLK

_f32 = jnp.float32


def _make_sc_agg_body(w_cols, fill_ones):
    def body(h_hbm, src_hbm, dst_hbm, dep_hbm, out_hbm,
             src_v, dst_v, rows_v, acc_sh, sem0, sem1):
        del dep_hbm
        c = lax.axis_index("c")
        s = lax.axis_index("s")
        w = c * NS + s

        def zrow(j, _):
            for k in range(w_cols // 16):
                rows_v[0, j, pl.ds(k * 16, 16)] = jnp.zeros((16,), _f32)
            return 0
        lax.fori_loop(0, CH, zrow, 0)
        for k in range(RPT // CH):
            pltpu.sync_copy(rows_v.at[0],
                            acc_sh.at[pl.ds(s * RPT + k * CH, CH)])

        plsc.subcore_barrier()

        HALF = NCHUNK // 2
        if fill_ones:
            def orow(j, _):
                for k in range(w_cols // 16):
                    rows_v[0, j, pl.ds(k * 16, 16)] = jnp.ones((16,), _f32)
                return 0
            lax.fori_loop(0, CH, orow, 0)

        for half in range(2):
            pltpu.sync_copy(src_hbm.at[w, pl.ds(half * HALF, HALF)], src_v)
            pltpu.sync_copy(dst_hbm.at[w, pl.ds(half * HALF, HALF)], dst_v)

            if fill_ones:
                def step1(j, _):
                    pltpu.sync_copy(rows_v.at[0], acc_sh.at[dst_v.at[j]],
                                    add=True)
                    return 0
                lax.fori_loop(0, HALF, step1, 0)
                continue

            pltpu.async_copy(h_hbm.at[src_v.at[0]], rows_v.at[0], sem0)

            def step(jj, _):
                j0 = 2 * jj
                j1 = j0 + 1
                pltpu.async_copy(h_hbm.at[src_v.at[j1]], rows_v.at[1], sem1)
                pltpu.make_async_copy(
                    h_hbm.at[src_v.at[j0]], rows_v.at[0], sem0).wait()
                pltpu.sync_copy(rows_v.at[0], acc_sh.at[dst_v.at[j0]],
                                add=True)

                @pl.when(jj < HALF // 2 - 1)
                def _():
                    pltpu.async_copy(h_hbm.at[src_v.at[j0 + 2]],
                                     rows_v.at[0], sem0)

                pltpu.make_async_copy(
                    h_hbm.at[src_v.at[j1]], rows_v.at[1], sem1).wait()
                pltpu.sync_copy(rows_v.at[1], acc_sh.at[dst_v.at[j1]],
                                add=True)
                return 0

            lax.fori_loop(0, HALF // 2, step, 0)

        plsc.subcore_barrier()

        for k in range(RPT // CH):
            rows = pl.ds(s * RPT + k * CH, CH)
            pltpu.sync_copy(acc_sh.at[rows], out_hbm.at[c, rows])

    return body


def _sc_agg(table, src_slabs, dst_slabs, dep, w_cols, fill_ones=False):
    mesh = plsc.VectorSubcoreMesh(
        core_axis_name="c", subcore_axis_name="s", num_cores=NC,
        num_subcores=NS)
    kern = pl.kernel(
        _make_sc_agg_body(w_cols, fill_ones),
        out_type=jax.ShapeDtypeStruct((NC, NPAD, w_cols), _f32),
        mesh=mesh,
        scratch_types=[
            pltpu.VMEM((NCHUNK // 2, CH), jnp.int32),
            pltpu.VMEM((NCHUNK // 2, CH), jnp.int32),
            pltpu.VMEM((2, CH, w_cols), _f32),
            pltpu.VMEM_SHARED((NPAD, w_cols), _f32),
            pltpu.SemaphoreType.DMA,
            pltpu.SemaphoreType.DMA,
        ],
    )
    return kern(table, src_slabs, dst_slabs, dep)


def _ln_relu(y, g, be):
    mu = jnp.mean(y, axis=1, keepdims=True)
    var = jnp.mean((y - mu) ** 2, axis=1, keepdims=True)
    yn = (y - mu) * lax.rsqrt(var + 1e-5) * g + be
    return jnp.maximum(yn, 0.0)


def _tc_in_body(x_ref, w_ref, b_ref, g_ref, be_ref, o_ref):
    y = jnp.dot(x_ref[...], w_ref[...], preferred_element_type=_f32)
    o_ref[...] = _ln_relu(y + b_ref[...], g_ref[...], be_ref[...])


def _tc_input(x_pad, w0t, b0, g0, be0):
    row = lambda i: (i, 0)
    fixed = lambda i: (0, 0)
    return pl.pallas_call(
        _tc_in_body,
        grid=(NBLK,),
        in_specs=[
            pl.BlockSpec((BLK, H), row),
            pl.BlockSpec((H, H), fixed),
            pl.BlockSpec((1, H), fixed),
            pl.BlockSpec((1, H), fixed),
            pl.BlockSpec((1, H), fixed),
        ],
        out_specs=pl.BlockSpec((BLK, H), row),
        out_shape=jax.ShapeDtypeStruct((NPAD, H), _f32),
    )(x_pad, w0t, b0, g0, be0)


def _tc_layer_body(p_ref, deg_ref, h_ref, wl_ref, wr_ref, b_ref, g_ref,
                   be_ref, o_ref):
    rdeg = 1.0 / jnp.maximum(deg_ref[0, :, :1] + deg_ref[1, :, :1], 1.0)
    agg = (p_ref[0] + p_ref[1]) * rdeg
    y = (jnp.dot(agg, wl_ref[...], preferred_element_type=_f32)
         + jnp.dot(h_ref[...].astype(_f32), wr_ref[...],
                   preferred_element_type=_f32)
         + b_ref[...])
    o_ref[...] = _ln_relu(y, g_ref[...], be_ref[...])


def _tc_layer(p, deg16, h, wlt, wrt, b, g, be):
    row = lambda i: (i, 0)
    fixed = lambda i: (0, 0)
    return pl.pallas_call(
        _tc_layer_body,
        grid=(NBLK,),
        in_specs=[
            pl.BlockSpec((NC, BLK, H), lambda i: (0, i, 0)),
            pl.BlockSpec((NC, BLK, 16), lambda i: (0, i, 0)),
            pl.BlockSpec((BLK, H), row),
            pl.BlockSpec((H, H), fixed),
            pl.BlockSpec((H, H), fixed),
            pl.BlockSpec((1, H), fixed),
            pl.BlockSpec((1, H), fixed),
            pl.BlockSpec((1, H), fixed),
        ],
        out_specs=pl.BlockSpec((BLK, H), row),
        out_shape=jax.ShapeDtypeStruct((NPAD, H), _f32),
    )(p, deg16, h, wlt, wrt, b, g, be)


def _tc_final_body(p_ref, deg_ref, h_ref, wl_ref, wr_ref, b_ref, g_ref,
                   be_ref, batch_ref, o_ref, mean_ref, max_ref, cnt_ref):
    rdeg = 1.0 / jnp.maximum(deg_ref[0, :, :1] + deg_ref[1, :, :1], 1.0)
    agg = (p_ref[0] + p_ref[1]) * rdeg
    y = (jnp.dot(agg, wl_ref[...], preferred_element_type=_f32)
         + jnp.dot(h_ref[...].astype(_f32), wr_ref[...],
                   preferred_element_type=_f32)
         + b_ref[...])
    ne = _ln_relu(y, g_ref[...], be_ref[...])
    o_ref[...] = ne

    i = pl.program_id(0)

    @pl.when(i == 0)
    def _():
        mean_ref[...] = jnp.zeros((G, H), _f32)
        cnt_ref[...] = jnp.zeros((G, H), _f32)
        max_ref[...] = jnp.full((G, H), -jnp.inf, _f32)

    b_ids = batch_ref[...]
    oh = (lax.broadcasted_iota(jnp.int32, (BLK, G), 1) == b_ids).astype(_f32)
    dn = (((0,), (0,)), ((), ()))
    mean_ref[...] += lax.dot_general(oh, ne, dn, preferred_element_type=_f32)
    cnt_ref[...] += lax.dot_general(oh, jnp.ones((BLK, H), _f32), dn,
                                    preferred_element_type=_f32)
    for gi in range(G):
        vg = jnp.where(b_ids == gi, ne, -jnp.inf)
        max_ref[gi:gi + 1, :] = jnp.maximum(
            max_ref[gi:gi + 1, :], jnp.max(vg, axis=0, keepdims=True))

    @pl.when(i == NBLK - 1)
    def _():
        mean_ref[...] = mean_ref[...] / jnp.maximum(cnt_ref[...], 1.0)


def _tc_final(p, deg16, h, wlt, wrt, b, g, be, batch_col):
    row = lambda i: (i, 0)
    fixed = lambda i: (0, 0)
    return pl.pallas_call(
        _tc_final_body,
        grid=(NBLK,),
        in_specs=[
            pl.BlockSpec((NC, BLK, H), lambda i: (0, i, 0)),
            pl.BlockSpec((NC, BLK, 16), lambda i: (0, i, 0)),
            pl.BlockSpec((BLK, H), row),
            pl.BlockSpec((H, H), fixed),
            pl.BlockSpec((H, H), fixed),
            pl.BlockSpec((1, H), fixed),
            pl.BlockSpec((1, H), fixed),
            pl.BlockSpec((1, H), fixed),
            pl.BlockSpec((BLK, 1), row),
        ],
        out_specs=[
            pl.BlockSpec((BLK, H), row),
            pl.BlockSpec((G, H), fixed),
            pl.BlockSpec((G, H), fixed),
        ],
        out_shape=[
            jax.ShapeDtypeStruct((NPAD, H), _f32),
            jax.ShapeDtypeStruct((G, H), _f32),
            jax.ShapeDtypeStruct((G, H), _f32),
        ],
        scratch_shapes=[pltpu.VMEM((G, H), _f32)],
    )(p, deg16, h, wlt, wrt, b, g, be, batch_col)


def kernel(x, edge_index, batch, params):
    src = edge_index[0]
    dst = edge_index[1]

    per_w = E // NW
    pad_per_w = EPW - per_w
    pad_idx = jnp.broadcast_to(
        N + jnp.arange(pad_per_w, dtype=jnp.int32)[None, :], (NW, pad_per_w))
    src_slabs = jnp.concatenate(
        [src.reshape(NW, per_w), pad_idx], axis=1).reshape(NW, NCHUNK, CH)
    dst_slabs = jnp.concatenate(
        [dst.reshape(NW, per_w), pad_idx], axis=1).reshape(NW, NCHUNK, CH)

    x_pad = jnp.zeros((NPAD, H), _f32).at[:N, :D_IN].set(x)
    batch_col = jnp.full((NPAD, 1), G, jnp.int32).at[:N, 0].set(batch)

    p = params
    w0t = jnp.zeros((H, H), _f32).at[:D_IN, :].set(p['W0'].T)
    r2 = lambda v: v.reshape(1, H)

    ones_tab = jnp.ones((NPAD, H), _f32)
    deg16 = _sc_agg(ones_tab, src_slabs, dst_slabs, dst_slabs, H,
                    fill_ones=True)[:, :, :16]

    h = _tc_input(x_pad, w0t, r2(p['b0']), r2(p['g0']), r2(p['be0']))

    for li in ('1', '2'):
        agg = _sc_agg(h, src_slabs, dst_slabs, deg16, H)
        h = _tc_layer(agg, deg16, h, p['Wl' + li].T, p['Wr' + li].T,
                      r2(p['bl' + li]), r2(p['g' + li]), r2(p['be' + li]))

    agg = _sc_agg(h, src_slabs, dst_slabs, deg16, H)
    node_pad, h_mean, h_max = _tc_final(
        agg, deg16, h, p['Wl3'].T, p['Wr3'].T,
        r2(p['bl3']), r2(p['g3']), r2(p['be3']), batch_col)

    node_embed = node_pad[:N]
    graph_embed = jnp.concatenate([h_mean, h_max], axis=-1)
    return node_embed, graph_embed
